# baseline (speedup 1.0000x reference)
import functools
import jax
import jax.numpy as jnp
from jax import lax
from jax.experimental import pallas as pl
from jax.experimental.pallas import tpu as pltpu
from jax.experimental.pallas import tpu_sc as plsc

N_NODES = 10000
N_EDGES = 320000
D_FEAT = 128
LANES = 16
N_WORKERS = 32
PER_W = N_EDGES // N_WORKERS
D_WORDS = D_FEAT // 2
CHUNK = 80
N_CHUNKS = PER_W // CHUNK
GROUPS = CHUNK // LANES
UNROLL = 16


def _edge_dot_body(z_hbm, src_hbm, dst_hbm, out_hbm,
                   sidx, didx, sbuf0, dbuf0, sbuf1, dbuf1, out_v, zsh,
                   sem0, sem1):
    sid = lax.axis_index("s")
    wid = lax.axis_index("c") * 16 + sid
    base = wid * PER_W
    rows = 624
    pltpu.sync_copy(z_hbm.at[pl.ds(sid * rows, rows)],
                    zsh.at[pl.ds(sid * rows, rows)])

    @pl.when(sid == 0)
    def _stage_tail():
        pltpu.sync_copy(z_hbm.at[pl.ds(16 * rows, N_NODES - 16 * rows)],
                        zsh.at[pl.ds(16 * rows, N_NODES - 16 * rows)])
    pltpu.sync_copy(src_hbm.at[pl.ds(base, PER_W)], sidx)
    pltpu.sync_copy(dst_hbm.at[pl.ds(base, PER_W)], didx)
    plsc.subcore_barrier()
    lane = lax.iota(jnp.int32, LANES)

    def start_gathers(c, sb, db, sem):
        off = c * CHUNK
        pltpu.async_copy(zsh.at[sidx.at[pl.ds(off, CHUNK)]], sb, sem)
        pltpu.async_copy(zsh.at[didx.at[pl.ds(off, CHUNK)]], db, sem)

    def wait_gathers(sb, db, sem):
        pltpu.make_async_copy(zsh.at[sidx.at[pl.ds(0, CHUNK)]], sb, sem).wait()
        pltpu.make_async_copy(zsh.at[didx.at[pl.ds(0, CHUNK)]], db, sem).wait()

    def compute_chunk(sb, db, obase):
        for g in range(GROUPS):
            rids = jnp.full((LANES,), g * LANES, jnp.int32) + lane

            def dbody(i, carry):
                acc, col = carry
                for _ in range(UNROLL):
                    sp = plsc.load_gather(sb, [rids, col])
                    tp = plsc.load_gather(db, [rids, col])
                    s_lo = plsc.bitcast(sp << 16, jnp.float32)
                    t_lo = plsc.bitcast(tp << 16, jnp.float32)
                    s_hi = plsc.bitcast(sp & jnp.int32(-65536), jnp.float32)
                    t_hi = plsc.bitcast(tp & jnp.int32(-65536), jnp.float32)
                    acc = acc + s_lo * t_lo + s_hi * t_hi
                    col = (col + 1) & (D_WORDS - 1)
                return acc, col

            acc, _ = lax.fori_loop(
                0, D_WORDS // UNROLL, dbody,
                (jnp.zeros((LANES,), jnp.float32), lane),
            )
            out_v[pl.ds(obase + g * LANES, LANES)] = 1.0 / (1.0 + jnp.exp(-acc))

    start_gathers(0, sbuf0, dbuf0, sem0)
    start_gathers(1, sbuf1, dbuf1, sem1)

    def pipe_body(j, carry):
        k2 = j * 2
        wait_gathers(sbuf0, dbuf0, sem0)
        compute_chunk(sbuf0, dbuf0, k2 * CHUNK)
        start_gathers(k2 + 2, sbuf0, dbuf0, sem0)
        wait_gathers(sbuf1, dbuf1, sem1)
        compute_chunk(sbuf1, dbuf1, (k2 + 1) * CHUNK)
        start_gathers(jnp.minimum(k2 + 3, N_CHUNKS - 1), sbuf1, dbuf1, sem1)
        return carry

    lax.fori_loop(0, (N_CHUNKS - 1) // 2, pipe_body, 0)
    wait_gathers(sbuf0, dbuf0, sem0)
    compute_chunk(sbuf0, dbuf0, (N_CHUNKS - 1) * CHUNK)
    wait_gathers(sbuf1, dbuf1, sem1)

    pltpu.sync_copy(out_v, out_hbm.at[pl.ds(base, PER_W)])


@functools.partial(
    pl.kernel,
    out_type=jax.ShapeDtypeStruct((N_EDGES,), jnp.float32),
    mesh=plsc.VectorSubcoreMesh(core_axis_name="c", subcore_axis_name="s"),
    compiler_params=pltpu.CompilerParams(
        needs_layout_passes=False, use_tc_tiling_on_sc=False
    ),
    scratch_types=[
        pltpu.VMEM((PER_W,), jnp.int32),
        pltpu.VMEM((PER_W,), jnp.int32),
        pltpu.VMEM((CHUNK, D_WORDS), jnp.int32),
        pltpu.VMEM((CHUNK, D_WORDS), jnp.int32),
        pltpu.VMEM((CHUNK, D_WORDS), jnp.int32),
        pltpu.VMEM((CHUNK, D_WORDS), jnp.int32),
        pltpu.VMEM((PER_W,), jnp.float32),
        pltpu.VMEM_SHARED((N_NODES, D_WORDS), jnp.int32),
        pltpu.SemaphoreType.DMA,
        pltpu.SemaphoreType.DMA,
    ],
)
def _edge_dot(z_hbm, src_hbm, dst_hbm, out_hbm,
              sidx, didx, sbuf0, dbuf0, sbuf1, dbuf1, out_v, zsh, sem0, sem1):
    _edge_dot_body(z_hbm, src_hbm, dst_hbm, out_hbm,
                   sidx, didx, sbuf0, dbuf0, sbuf1, dbuf1, out_v, zsh, sem0, sem1)


def kernel(z, edge_index, weights):
    ei = edge_index.astype(jnp.int32)
    zp = lax.bitcast_convert_type(
        z.astype(jnp.bfloat16).reshape(N_NODES, D_WORDS, 2), jnp.int32
    )
    return _edge_dot(zp, ei[0], ei[1])


# speedup vs baseline: 1.3439x; 1.3439x over previous
import functools
import jax
import jax.numpy as jnp
from jax import lax
from jax.experimental import pallas as pl
from jax.experimental.pallas import tpu as pltpu
from jax.experimental.pallas import tpu_sc as plsc

N_NODES = 10000
N_EDGES = 320000
D_FEAT = 128
LANES = 16
N_WORKERS = 32
PER_W = N_EDGES // N_WORKERS
CHUNK = 80
N_CHUNKS = PER_W // CHUNK
GROUPS = CHUNK // LANES
UNROLL = 16


def _edge_dot_body(z_hbm, src_hbm, dst_hbm, out_hbm,
                   sidx, didx,
                   sbuf0, dbuf0, sbuf1, dbuf1, sbuf2, dbuf2, sbuf3, dbuf3,
                   out_v, sem0, sem1, sem2, sem3):
    sbufs = (sbuf0, sbuf1, sbuf2, sbuf3)
    dbufs = (dbuf0, dbuf1, dbuf2, dbuf3)
    sems = (sem0, sem1, sem2, sem3)
    wid = lax.axis_index("c") * 16 + lax.axis_index("s")
    base = wid * PER_W
    pltpu.sync_copy(src_hbm.at[pl.ds(base, PER_W)], sidx)
    pltpu.sync_copy(dst_hbm.at[pl.ds(base, PER_W)], didx)
    lane = lax.iota(jnp.int32, LANES)

    def start_gathers(c, sb, db, sem):
        off = c * CHUNK
        pltpu.async_copy(z_hbm.at[sidx.at[pl.ds(off, CHUNK)]], sb, sem)
        pltpu.async_copy(z_hbm.at[didx.at[pl.ds(off, CHUNK)]], db, sem)

    def wait_gathers(sb, db, sem):
        pltpu.make_async_copy(z_hbm.at[sidx.at[pl.ds(0, CHUNK)]], sb, sem).wait()
        pltpu.make_async_copy(z_hbm.at[didx.at[pl.ds(0, CHUNK)]], db, sem).wait()

    def compute_chunk(sb, db, obase):
        for g in range(GROUPS):
            rids = jnp.full((LANES,), g * LANES, jnp.int32) + lane

            def dbody(i, carry):
                acc, col = carry
                for _ in range(UNROLL):
                    s = plsc.load_gather(sb, [rids, col])
                    t = plsc.load_gather(db, [rids, col])
                    acc = acc + s * t
                    col = (col + 1) & (D_FEAT - 1)
                return acc, col

            acc, _ = lax.fori_loop(
                0, D_FEAT // UNROLL, dbody,
                (jnp.zeros((LANES,), jnp.float32), lane),
            )
            out_v[pl.ds(obase + g * LANES, LANES)] = 1.0 / (1.0 + jnp.exp(-acc))

    for b in range(4):
        start_gathers(b, sbufs[b], dbufs[b], sems[b])

    def pipe_body(j, carry):
        k4 = j * 4
        for b in range(4):
            wait_gathers(sbufs[b], dbufs[b], sems[b])
            compute_chunk(sbufs[b], dbufs[b], (k4 + b) * CHUNK)
            start_gathers(k4 + b + 4, sbufs[b], dbufs[b], sems[b])
        return carry

    lax.fori_loop(0, (N_CHUNKS - 5) // 4, pipe_body, 0)
    wait_gathers(sbuf0, dbuf0, sem0)
    compute_chunk(sbuf0, dbuf0, 120 * CHUNK)
    start_gathers(124, sbuf0, dbuf0, sem0)
    wait_gathers(sbuf1, dbuf1, sem1)
    compute_chunk(sbuf1, dbuf1, 121 * CHUNK)
    wait_gathers(sbuf2, dbuf2, sem2)
    compute_chunk(sbuf2, dbuf2, 122 * CHUNK)
    wait_gathers(sbuf3, dbuf3, sem3)
    compute_chunk(sbuf3, dbuf3, 123 * CHUNK)
    wait_gathers(sbuf0, dbuf0, sem0)
    compute_chunk(sbuf0, dbuf0, 124 * CHUNK)

    pltpu.sync_copy(out_v, out_hbm.at[pl.ds(base, PER_W)])


@functools.partial(
    pl.kernel,
    out_type=jax.ShapeDtypeStruct((N_EDGES,), jnp.float32),
    mesh=plsc.VectorSubcoreMesh(core_axis_name="c", subcore_axis_name="s"),
    compiler_params=pltpu.CompilerParams(needs_layout_passes=False),
    scratch_types=[
        pltpu.VMEM((PER_W,), jnp.int32),
        pltpu.VMEM((PER_W,), jnp.int32),
        pltpu.VMEM((CHUNK, D_FEAT), jnp.float32),
        pltpu.VMEM((CHUNK, D_FEAT), jnp.float32),
        pltpu.VMEM((CHUNK, D_FEAT), jnp.float32),
        pltpu.VMEM((CHUNK, D_FEAT), jnp.float32),
        pltpu.VMEM((CHUNK, D_FEAT), jnp.float32),
        pltpu.VMEM((CHUNK, D_FEAT), jnp.float32),
        pltpu.VMEM((CHUNK, D_FEAT), jnp.float32),
        pltpu.VMEM((CHUNK, D_FEAT), jnp.float32),
        pltpu.VMEM((PER_W,), jnp.float32),
        pltpu.SemaphoreType.DMA,
        pltpu.SemaphoreType.DMA,
        pltpu.SemaphoreType.DMA,
        pltpu.SemaphoreType.DMA,
    ],
)
def _edge_dot(z_hbm, src_hbm, dst_hbm, out_hbm,
              sidx, didx,
              sbuf0, dbuf0, sbuf1, dbuf1, sbuf2, dbuf2, sbuf3, dbuf3,
              out_v, sem0, sem1, sem2, sem3):
    _edge_dot_body(z_hbm, src_hbm, dst_hbm, out_hbm,
                   sidx, didx,
                   sbuf0, dbuf0, sbuf1, dbuf1, sbuf2, dbuf2, sbuf3, dbuf3,
                   out_v, sem0, sem1, sem2, sem3)


def kernel(z, edge_index, weights):
    ei = edge_index.astype(jnp.int32)
    return _edge_dot(z, ei[0], ei[1])


# speedup vs baseline: 1.3881x; 1.0329x over previous
import functools
import jax
import jax.numpy as jnp
from jax import lax
from jax.experimental import pallas as pl
from jax.experimental.pallas import tpu as pltpu
from jax.experimental.pallas import tpu_sc as plsc

N_NODES = 10000
N_EDGES = 320000
D_FEAT = 128
LANES = 16
N_WORKERS = 32
PER_W = N_EDGES // N_WORKERS
CHUNK = 80
N_CHUNKS = PER_W // CHUNK
GROUPS = CHUNK // LANES
UNROLL = 16


def _edge_dot_body(z_hbm, src_hbm, dst_hbm, out_hbm,
                   sidx, didx, sbuf0, dbuf0, sbuf1, dbuf1, out_v,
                   sem0, sem1):
    wid = lax.axis_index("c") * 16 + lax.axis_index("s")
    base = wid * PER_W
    pltpu.sync_copy(src_hbm.at[pl.ds(base, PER_W)], sidx)
    pltpu.sync_copy(dst_hbm.at[pl.ds(base, PER_W)], didx)
    lane = lax.iota(jnp.int32, LANES)

    def start_gathers(c, sb, db, sem):
        off = c * CHUNK
        pltpu.async_copy(z_hbm.at[sidx.at[pl.ds(off, CHUNK)]], sb, sem)
        pltpu.async_copy(z_hbm.at[didx.at[pl.ds(off, CHUNK)]], db, sem)

    def wait_gathers(sb, db, sem):
        pltpu.make_async_copy(z_hbm.at[sidx.at[pl.ds(0, CHUNK)]], sb, sem).wait()
        pltpu.make_async_copy(z_hbm.at[didx.at[pl.ds(0, CHUNK)]], db, sem).wait()

    def compute_chunk(sb, db, obase):
        for g in range(GROUPS):
            rids = jnp.full((LANES,), g * LANES, jnp.int32) + lane

            def dbody(i, carry):
                acc, col = carry
                for _ in range(UNROLL):
                    s = plsc.load_gather(sb, [rids, col])
                    t = plsc.load_gather(db, [rids, col])
                    acc = acc + s * t
                    col = (col + 1) & (D_FEAT - 1)
                return acc, col

            acc, _ = lax.fori_loop(
                0, D_FEAT // UNROLL, dbody,
                (jnp.zeros((LANES,), jnp.float32), lane),
            )
            out_v[pl.ds(obase + g * LANES, LANES)] = 1.0 / (1.0 + jnp.exp(-acc))

    start_gathers(0, sbuf0, dbuf0, sem0)
    start_gathers(1, sbuf1, dbuf1, sem1)

    def pipe_body(j, carry):
        k2 = j * 2
        wait_gathers(sbuf0, dbuf0, sem0)
        compute_chunk(sbuf0, dbuf0, k2 * CHUNK)
        start_gathers(k2 + 2, sbuf0, dbuf0, sem0)
        wait_gathers(sbuf1, dbuf1, sem1)
        compute_chunk(sbuf1, dbuf1, (k2 + 1) * CHUNK)
        start_gathers(jnp.minimum(k2 + 3, N_CHUNKS - 1), sbuf1, dbuf1, sem1)
        return carry

    lax.fori_loop(0, (N_CHUNKS - 1) // 2, pipe_body, 0)
    wait_gathers(sbuf0, dbuf0, sem0)
    compute_chunk(sbuf0, dbuf0, (N_CHUNKS - 1) * CHUNK)
    wait_gathers(sbuf1, dbuf1, sem1)

    pltpu.sync_copy(out_v, out_hbm.at[pl.ds(base, PER_W)])


@functools.partial(
    pl.kernel,
    out_type=jax.ShapeDtypeStruct((N_EDGES,), jnp.float32),
    mesh=plsc.VectorSubcoreMesh(core_axis_name="c", subcore_axis_name="s"),
    compiler_params=pltpu.CompilerParams(needs_layout_passes=False),
    scratch_types=[
        pltpu.VMEM((PER_W,), jnp.int32),
        pltpu.VMEM((PER_W,), jnp.int32),
        pltpu.VMEM((CHUNK, D_FEAT), jnp.float32),
        pltpu.VMEM((CHUNK, D_FEAT), jnp.float32),
        pltpu.VMEM((CHUNK, D_FEAT), jnp.float32),
        pltpu.VMEM((CHUNK, D_FEAT), jnp.float32),
        pltpu.VMEM((PER_W,), jnp.float32),
        pltpu.SemaphoreType.DMA,
        pltpu.SemaphoreType.DMA,
    ],
)
def _edge_dot(z_hbm, src_hbm, dst_hbm, out_hbm,
              sidx, didx, sbuf0, dbuf0, sbuf1, dbuf1, out_v, sem0, sem1):
    _edge_dot_body(z_hbm, src_hbm, dst_hbm, out_hbm,
                   sidx, didx, sbuf0, dbuf0, sbuf1, dbuf1, out_v, sem0, sem1)


def kernel(z, edge_index, weights):
    ei = edge_index.astype(jnp.int32)
    return _edge_dot(z, ei[0], ei[1])
